# baseline (speedup 1.0000x reference)
import functools

import jax
import jax.numpy as jnp
from jax import lax
from jax.experimental import pallas as pl
from jax.experimental.pallas import tpu as pltpu
from jax.experimental.pallas import tpu_sc as plsc

P = 120000
C = 64
GX = 432
GY = 496
TOT = GX * GY
NB = 4
NC = 2
NS = 16
NWORK = NC * NS
K = NB * TOT // NWORK
CHUNK = GX
NCHUNK = K // CHUNK
PPS = P // NS
HPS = PPS // 2
NVEC = (HPS + 15) // 16
SEG_CAP = 8192
SEGC = 2048
PB = 112
ROWS_W = K // 2
ZR = CHUNK // 2
SENT = 2 ** 30

_mesh = plsc.VectorSubcoreMesh(core_axis_name="c", subcore_axis_name="s")


@functools.partial(
    pl.kernel,
    mesh=_mesh,
    compiler_params=pltpu.CompilerParams(use_tc_tiling_on_sc=False,
                                         needs_layout_passes=False),
    out_type=jax.ShapeDtypeStruct((NB * TOT // 2, 2 * C), jnp.float32),
    scratch_types=[
        pltpu.VMEM((HPS, 4), jnp.int32),
        pltpu.VMEM((SEG_CAP,), jnp.int32),
        pltpu.VMEM((SEG_CAP,), jnp.int32),
        pltpu.VMEM((K,), jnp.int32),
        pltpu.VMEM((ZR, 2 * C), jnp.float32),
        pltpu.VMEM((SEGC,), jnp.int32),
        pltpu.VMEM((SEGC,), jnp.int32),
        pltpu.VMEM((PB + 16,), jnp.int32),
        pltpu.VMEM((PB + 16,), jnp.int32),
        pltpu.VMEM((PB + 16, C), jnp.float32),
        pltpu.VMEM((16,), jnp.int32),
        pltpu.VMEM((NS, 16), jnp.int32),
        pltpu.VMEM_SHARED((NS, SEG_CAP), jnp.int32),
        pltpu.VMEM_SHARED((NS, SEG_CAP), jnp.int32),
        pltpu.VMEM_SHARED((NS, 16), jnp.int32),
        pltpu.SemaphoreType.DMA,
        pltpu.SemaphoreType.DMA,
    ],
)
def _pp_scatter(x0_hbm, x1_hbm, out_hbm, coords_v, skey_v, spid_v, map_v,
                zero_v, segk_v, segp_v, widx_v, wpid_v, rows_v,
                stage_v, counts_rd, segk_sh, segp_sh, counts_sh, sem_zero,
                sem_patch):
    cid = lax.axis_index("c")
    sid = lax.axis_index("s")
    w = sid * NC + cid
    lanes = lax.iota(jnp.int32, 16)
    zvec = jnp.zeros((16,), jnp.int32)
    sentv = jnp.full((16,), SENT, jnp.int32)

    @pl.loop(0, ZR)
    def _(r):
        @pl.loop(0, 2 * C, step=16)
        def _(cc):
            zero_v[r, pl.ds(cc, 16)] = jnp.zeros((16,), jnp.float32)

    @pl.loop(0, NCHUNK)
    def _(j):
        pltpu.async_copy(
            zero_v,
            out_hbm.at[pl.ds(w * ROWS_W + j * ZR, ZR), :],
            sem_zero)

    @pl.loop(0, SEG_CAP, step=16)
    def _(i):
        skey_v[pl.ds(i, 16)] = sentv

    cnt = jnp.int32(0)
    for h in range(2):
        pltpu.sync_copy(
            x1_hbm.at[pl.ds(pl.multiple_of(sid * PPS + h * HPS, 2), HPS)],
            coords_v)

        def scan_body(v, cnt, h=h):
            base = v * 16
            idx = base + lanes
            valid = idx < HPS
            ii = jnp.where(valid, idx, 0)
            b = plsc.load_gather(coords_v, [ii, zvec], mask=valid)
            c1 = plsc.load_gather(coords_v, [ii, zvec + 1], mask=valid)
            c2 = plsc.load_gather(coords_v, [ii, zvec + 2], mask=valid)
            c3 = plsc.load_gather(coords_v, [ii, zvec + 3], mask=valid)
            m = valid & (b < NB)
            key = b * TOT + c1 + c2 * GX + c3
            nm = jnp.sum(m.astype(jnp.int32))

            def compact(cnt):
                cnt = pl.multiple_of(cnt, 8)
                plsc.store_compressed(skey_v.at[pl.ds(cnt, 16)], key, mask=m)
                plsc.store_compressed(spid_v.at[pl.ds(cnt, 16)],
                                      sid * PPS + h * HPS + idx, mask=m)
                return (cnt + nm + 7) & (-8)

            return lax.cond(nm > 0, compact, lambda c: c, cnt)

        cnt = lax.fori_loop(0, NVEC, scan_body, cnt)

    stage_v[...] = zvec + cnt
    pltpu.sync_copy(stage_v, counts_sh.at[sid])
    pltpu.sync_copy(skey_v, segk_sh.at[sid])
    pltpu.sync_copy(spid_v, segp_sh.at[sid])
    plsc.subcore_barrier()

    @pl.loop(0, K, step=16)
    def _(i):
        map_v[pl.ds(i, 16)] = jnp.zeros((16,), jnp.int32)

    pltpu.sync_copy(counts_sh, counts_rd)
    cvec = plsc.load_gather(counts_rd, [lanes, zvec])
    lo = w * K

    for s2 in range(NS):
        cnt2 = cvec[s2]

        def chunk_cond(c0):
            return c0 < cnt2

        def chunk_body(c0, cnt2=cnt2, s2=s2):
            c0 = pl.multiple_of(c0, SEGC)
            pltpu.sync_copy(segk_sh.at[s2, pl.ds(c0, SEGC)], segk_v)
            pltpu.sync_copy(segp_sh.at[s2, pl.ds(c0, SEGC)], segp_v)
            n = jnp.minimum(cnt2 - c0, SEGC)

            def vec_body(vv, _):
                off = pl.multiple_of(vv * 16, 16)
                kv = segk_v[pl.ds(off, 16)]
                pv = segp_v[pl.ds(off, 16)]
                inr = ((off + lanes) < n) & (kv >= lo) & (kv < lo + K)
                lk = jnp.where(inr, kv - lo, 0)
                pw = pv + 1
                plsc.store_scatter(map_v, [lk], pw, mask=inr)
                got = plsc.load_gather(map_v, [lk], mask=inr)
                need = (inr & (got < pw)).astype(jnp.int32)

                def wcond(st):
                    return jnp.sum(st) > 0

                def wbody(st):
                    nb = st != 0
                    plsc.store_scatter(map_v, [lk], pw, mask=nb)
                    g2 = plsc.load_gather(map_v, [lk], mask=nb)
                    return (nb & (g2 < pw)).astype(jnp.int32)

                lax.while_loop(wcond, wbody, need)
                return 0

            lax.fori_loop(0, (n + 15) // 16, vec_body, 0)
            return c0 + SEGC

        lax.while_loop(chunk_cond, chunk_body, jnp.int32(0))

    @pl.loop(0, NCHUNK)
    def _(j):
        pltpu.make_async_copy(
            zero_v,
            out_hbm.at[pl.ds(w * ROWS_W, ZR), :],
            sem_zero).wait()

    @pl.loop(0, PB + 16, step=16)
    def _(i):
        wpid_v[pl.ds(i, 16)] = i + lanes
        widx_v[pl.ds(i, 16)] = zvec - 1

    def flush(st):
        nb, nbt = st
        pltpu.sync_copy(x0_hbm.at[wpid_v], rows_v)

        def ob(i16v, _):
            i16 = pl.multiple_of(i16v * 16, 16)
            vv = widx_v[pl.ds(i16, 16)]
            for j in range(16):
                @pl.when((i16 + j < nb) & (vv[j] >= 0))
                def _(j=j, i16=i16, vv=vv):
                    gk = lo + vv[j]
                    moff = pl.multiple_of((gk & 1) * C, 8)
                    pltpu.async_copy(
                        rows_v.at[pl.ds(i16 + j, 1), :],
                        out_hbm.at[pl.ds(gk >> 1, 1), pl.ds(moff, C)],
                        sem_patch)
            return 0

        lax.fori_loop(0, (nb + 15) // 16, ob, 0)

        def dr_body(i, _):
            pltpu.make_async_copy(
                rows_v.at[pl.ds(0, 1), :],
                out_hbm.at[pl.ds(0, 1), pl.ds(0, C)],
                sem_patch).wait()
            return 0

        lax.fori_loop(0, nbt, dr_body, 0)

        @pl.loop(0, PB + 16, step=16)
        def _(i):
            widx_v[pl.ds(i, 16)] = zvec - 1
        return (jnp.int32(0), jnp.int32(0))

    def scan3(v, st):
        st = lax.cond(st[0] > PB - 16, flush, lambda s: s, st)
        mv = map_v[pl.ds(v * 16, 16)]
        mw = mv != 0
        nw = jnp.sum(mw.astype(jnp.int32))

        def haswin(st, v=v, mv=mv, mw=mw, nw=nw):
            nb, nbt = st
            nb = pl.multiple_of(nb, 8)
            plsc.store_compressed(widx_v.at[pl.ds(nb, 16)], v * 16 + lanes,
                                  mask=mw)
            plsc.store_compressed(wpid_v.at[pl.ds(nb, 16)], mv - 1, mask=mw)
            return ((nb + nw + 7) & (-8), nbt + nw)

        return lax.cond(nw > 0, haswin, lambda s: s, st)

    st = lax.fori_loop(0, K // 16, scan3, (jnp.int32(0), jnp.int32(0)))
    lax.cond(st[1] > 0, flush, lambda s: s, st)


def kernel(x_0, x_1, batchsize):
    out = _pp_scatter(x_0, x_1)
    return (out.reshape(NB, GY, GX // 2, 2, C)
            .transpose(0, 4, 1, 2, 3)
            .reshape(NB, C, GY, GX))


# speedup vs baseline: 2.7089x; 2.7089x over previous
import functools

import jax
import jax.numpy as jnp
from jax import lax
from jax.experimental import pallas as pl
from jax.experimental.pallas import tpu as pltpu
from jax.experimental.pallas import tpu_sc as plsc

P = 120000
C = 64
GX = 432
GY = 496
TOT = GX * GY
NB = 4
NC = 2
NS = 16
NWORK = NC * NS
K = NB * TOT // NWORK
CHUNK = GX
NCHUNK = K // CHUNK
PPS = P // NS
HPS = PPS // 2
NVEC = (HPS + 15) // 16
SEG_CAP = 8192
SEGC = 2048
PB = 112
SENT = 2 ** 30

_mesh = plsc.VectorSubcoreMesh(core_axis_name="c", subcore_axis_name="s")


@functools.partial(
    pl.kernel,
    mesh=_mesh,
    compiler_params=pltpu.CompilerParams(use_tc_tiling_on_sc=False,
                                         needs_layout_passes=False),
    out_type=jax.ShapeDtypeStruct((NB * TOT, C), jnp.float32),
    scratch_types=[
        pltpu.VMEM((HPS, 4), jnp.int32),
        pltpu.VMEM((SEG_CAP,), jnp.int32),
        pltpu.VMEM((SEG_CAP,), jnp.int32),
        pltpu.VMEM((K,), jnp.int32),
        pltpu.VMEM((CHUNK, C), jnp.float32),
        pltpu.VMEM((SEGC,), jnp.int32),
        pltpu.VMEM((SEGC,), jnp.int32),
        pltpu.VMEM((PB + 16,), jnp.int32),
        pltpu.VMEM((PB + 16,), jnp.int32),
        pltpu.VMEM((PB + 16, C), jnp.float32),
        pltpu.VMEM((16,), jnp.int32),
        pltpu.VMEM((NS, 16), jnp.int32),
        pltpu.VMEM_SHARED((NS, SEG_CAP), jnp.int32),
        pltpu.VMEM_SHARED((NS, SEG_CAP), jnp.int32),
        pltpu.VMEM_SHARED((NS, 16), jnp.int32),
        pltpu.SemaphoreType.DMA,
        pltpu.SemaphoreType.DMA,
    ],
)
def _pp_scatter(x0_hbm, x1_hbm, out_hbm, coords_v, skey_v, spid_v, map_v,
                zero_v, segk_v, segp_v, widx_v, wpid_v, rows_v,
                stage_v, counts_rd, segk_sh, segp_sh, counts_sh, sem_zero,
                sem_patch):
    cid = lax.axis_index("c")
    sid = lax.axis_index("s")
    w = sid * NC + cid
    lanes = lax.iota(jnp.int32, 16)
    zvec = jnp.zeros((16,), jnp.int32)
    sentv = jnp.full((16,), SENT, jnp.int32)

    @pl.loop(0, CHUNK)
    def _(r):
        @pl.loop(0, C, step=16)
        def _(cc):
            zero_v[r, pl.ds(cc, 16)] = jnp.zeros((16,), jnp.float32)

    @pl.loop(0, NCHUNK)
    def _(j):
        pltpu.async_copy(
            zero_v,
            out_hbm.at[pl.ds(w * K + j * CHUNK, CHUNK), :],
            sem_zero)

    @pl.loop(0, SEG_CAP, step=16)
    def _(i):
        skey_v[pl.ds(i, 16)] = sentv

    cnt = jnp.int32(0)
    for h in range(2):
        pltpu.sync_copy(
            x1_hbm.at[pl.ds(pl.multiple_of(sid * PPS + h * HPS, 2), HPS)],
            coords_v)

        def scan_body(v, cnt, h=h):
            base = v * 16
            idx = base + lanes
            valid = idx < HPS
            ii = jnp.where(valid, idx, 0)
            b = plsc.load_gather(coords_v, [ii, zvec], mask=valid)
            c1 = plsc.load_gather(coords_v, [ii, zvec + 1], mask=valid)
            c2 = plsc.load_gather(coords_v, [ii, zvec + 2], mask=valid)
            c3 = plsc.load_gather(coords_v, [ii, zvec + 3], mask=valid)
            m = valid & (b < NB)
            key = b * TOT + c1 + c2 * GX + c3
            nm = jnp.sum(m.astype(jnp.int32))

            def compact(cnt):
                cnt = pl.multiple_of(cnt, 8)
                plsc.store_compressed(skey_v.at[pl.ds(cnt, 16)], key, mask=m)
                plsc.store_compressed(spid_v.at[pl.ds(cnt, 16)],
                                      sid * PPS + h * HPS + idx, mask=m)
                return (cnt + nm + 7) & (-8)

            return lax.cond(nm > 0, compact, lambda c: c, cnt)

        cnt = lax.fori_loop(0, NVEC, scan_body, cnt)

    stage_v[...] = zvec + cnt
    pltpu.sync_copy(stage_v, counts_sh.at[sid])
    pltpu.sync_copy(skey_v, segk_sh.at[sid])
    pltpu.sync_copy(spid_v, segp_sh.at[sid])
    plsc.subcore_barrier()

    @pl.loop(0, K, step=16)
    def _(i):
        map_v[pl.ds(i, 16)] = jnp.zeros((16,), jnp.int32)

    pltpu.sync_copy(counts_sh, counts_rd)
    cvec = plsc.load_gather(counts_rd, [lanes, zvec])
    lo = w * K

    for s2 in range(NS):
        cnt2 = cvec[s2]

        def chunk_cond(c0):
            return c0 < cnt2

        def chunk_body(c0, cnt2=cnt2, s2=s2):
            c0 = pl.multiple_of(c0, SEGC)
            pltpu.sync_copy(segk_sh.at[s2, pl.ds(c0, SEGC)], segk_v)
            pltpu.sync_copy(segp_sh.at[s2, pl.ds(c0, SEGC)], segp_v)
            n = jnp.minimum(cnt2 - c0, SEGC)

            def vec_body(vv, _):
                off = pl.multiple_of(vv * 16, 16)
                kv = segk_v[pl.ds(off, 16)]
                pv = segp_v[pl.ds(off, 16)]
                inr = ((off + lanes) < n) & (kv >= lo) & (kv < lo + K)
                lk = jnp.where(inr, kv - lo, 0)
                pw = pv + 1
                plsc.store_scatter(map_v, [lk], pw, mask=inr)
                got = plsc.load_gather(map_v, [lk], mask=inr)
                need = (inr & (got < pw)).astype(jnp.int32)

                def wcond(st):
                    return jnp.sum(st) > 0

                def wbody(st):
                    nb = st != 0
                    plsc.store_scatter(map_v, [lk], pw, mask=nb)
                    g2 = plsc.load_gather(map_v, [lk], mask=nb)
                    return (nb & (g2 < pw)).astype(jnp.int32)

                lax.while_loop(wcond, wbody, need)
                return 0

            lax.fori_loop(0, (n + 15) // 16, vec_body, 0)
            return c0 + SEGC

        lax.while_loop(chunk_cond, chunk_body, jnp.int32(0))

    @pl.loop(0, NCHUNK)
    def _(j):
        pltpu.make_async_copy(
            zero_v,
            out_hbm.at[pl.ds(w * K, CHUNK), :],
            sem_zero).wait()

    @pl.loop(0, PB + 16, step=16)
    def _(i):
        wpid_v[pl.ds(i, 16)] = i + lanes
        widx_v[pl.ds(i, 16)] = zvec - 1

    def flush(st):
        nb, nbt = st
        pltpu.sync_copy(x0_hbm.at[wpid_v], rows_v)

        def ob(i16v, _):
            i16 = pl.multiple_of(i16v * 16, 16)
            vv = widx_v[pl.ds(i16, 16)]
            for j in range(16):
                @pl.when((i16 + j < nb) & (vv[j] >= 0))
                def _(j=j, i16=i16, vv=vv):
                    pltpu.async_copy(
                        rows_v.at[pl.ds(i16 + j, 1), :],
                        out_hbm.at[pl.ds(lo + vv[j], 1), :],
                        sem_patch)
            return 0

        lax.fori_loop(0, (nb + 15) // 16, ob, 0)

        def dr_body(i, _):
            pltpu.make_async_copy(
                rows_v.at[pl.ds(0, 1), :],
                out_hbm.at[pl.ds(lo, 1), :],
                sem_patch).wait()
            return 0

        lax.fori_loop(0, nbt, dr_body, 0)

        @pl.loop(0, PB + 16, step=16)
        def _(i):
            widx_v[pl.ds(i, 16)] = zvec - 1
        return (jnp.int32(0), jnp.int32(0))

    def scan3(v, st):
        st = lax.cond(st[0] > PB - 16, flush, lambda s: s, st)
        mv = map_v[pl.ds(v * 16, 16)]
        mw = mv != 0
        nw = jnp.sum(mw.astype(jnp.int32))

        def haswin(st, v=v, mv=mv, mw=mw, nw=nw):
            nb, nbt = st
            nb = pl.multiple_of(nb, 8)
            plsc.store_compressed(widx_v.at[pl.ds(nb, 16)], v * 16 + lanes,
                                  mask=mw)
            plsc.store_compressed(wpid_v.at[pl.ds(nb, 16)], mv - 1, mask=mw)
            return ((nb + nw + 7) & (-8), nbt + nw)

        return lax.cond(nw > 0, haswin, lambda s: s, st)

    st = lax.fori_loop(0, K // 16, scan3, (jnp.int32(0), jnp.int32(0)))
    lax.cond(st[1] > 0, flush, lambda s: s, st)


def kernel(x_0, x_1, batchsize):
    out = _pp_scatter(x_0, x_1)
    return out.reshape(NB, GY, GX, C).transpose(0, 3, 1, 2)
